# baseline (speedup 1.0000x reference)
import jax
import jax.numpy as jnp
from jax.experimental import pallas as pl
from jax.experimental.pallas import tpu as pltpu

_PARALLEL = pltpu.CompilerParams(dimension_semantics=("parallel",))
_ARBITRARY = pltpu.CompilerParams(dimension_semantics=("arbitrary",))
_HALF = 256


def _pick_blk(m, pref):
    for b in (pref, 200, 100, 50, 25, 10, 8, 5, 4, 2, 1):
        if m % b == 0:
            return b
    return m


def _gc1_kernel(adj_ref, x_ref, w1_ref, b_ref, wa_ref, wb_ref,
                h_ref, qa_ref, qbt_ref, ve_ref, t_ref):
    @pl.when(pl.program_id(0) == 0)
    def _():
        t_ref[...] = jnp.dot(x_ref[...], w1_ref[...],
                             preferred_element_type=jnp.float32)

    acc = jnp.dot(adj_ref[...], t_ref[...],
                  preferred_element_type=jnp.float32)
    h = jnp.maximum(acc + b_ref[...], 0.0)
    h_ref[...] = h
    qa_ref[...] = jnp.dot(h, wa_ref[...],
                          preferred_element_type=jnp.float32
                          ).astype(jnp.bfloat16)
    qb = jnp.dot(h, wb_ref[...], preferred_element_type=jnp.float32)
    qbt_ref[...] = qb.T.astype(jnp.bfloat16)
    nhid = h_ref.shape[1]
    ve_ref[:, :nhid] = (h * 0.0625).astype(ve_ref.dtype)
    ve_ref[:, nhid:] = jnp.full_like(ve_ref[:, nhid:], 0.0625)


def _crf1_kernel(qa_ref, qbt_ref, ve_ref, h_ref, ab_ref, ve2_ref, p_ref):
    qbt = qbt_ref[...]
    ve = ve_ref[...]
    nhid = h_ref.shape[1]
    a = ab_ref[0, 0]
    b = ab_ref[0, 1]
    chain = _HALF if qa_ref.shape[0] % _HALF == 0 else qa_ref.shape[0]
    for half in range(qa_ref.shape[0] // chain):
        sl = pl.ds(half * chain, chain)
        logits = jnp.dot(qa_ref[sl, :], qbt,
                         preferred_element_type=jnp.float32
                         ).astype(jnp.bfloat16)
        m = jnp.max(logits, axis=1, keepdims=True)
        p = jnp.exp(logits - m).astype(p_ref.dtype)
        p_ref[sl, :] = p
        num = jnp.dot(p, ve, preferred_element_type=jnp.float32)
        s = num[:, nhid:nhid + 1]
        ve2_ref[sl, :nhid] = ((a * h_ref[sl, :] + b * (num[:, :nhid] / s))
                              * 0.0625).astype(ve2_ref.dtype)
    ve2_ref[:, nhid:] = jnp.full_like(ve2_ref[:, nhid:], 0.0625)


def _crf2_kernel(p_ref, ve_ref, h_ref, ab_ref, w2_ref, t2_ref):
    num = jnp.dot(p_ref[...], ve_ref[...],
                  preferred_element_type=jnp.float32)
    nhid = h_ref.shape[1]
    s = num[:, nhid:nhid + 1]
    a = ab_ref[0, 0]
    b = ab_ref[0, 1]
    h2 = a * h_ref[...] + b * (num[:, :nhid] / s)
    t2_ref[...] = jnp.dot(h2, w2_ref[...],
                          preferred_element_type=jnp.float32)


def _gc2_kernel(adj_ref, t2_ref, b_ref, out_ref):
    logits = jnp.dot(adj_ref[...], t2_ref[...],
                     preferred_element_type=jnp.float32) + b_ref[...]
    m = jnp.max(logits, axis=1, keepdims=True)
    ls = logits - m
    lse = jnp.log(jnp.sum(jnp.exp(ls), axis=1, keepdims=True))
    out_ref[...] = ls - lse


def kernel(x, adj, W1, b1, W2, b2, Wa, Wb, alpha, beta):
    n, nfeat = x.shape
    nhid = W1.shape[1]
    nclass = W2.shape[1]
    blk = 256 if n >= 256 else _pick_blk(n, 8)
    ggrid = pl.cdiv(n, blk)
    cblk = 2 * _HALF if n >= 2 * _HALF else _pick_blk(n, 8)
    cgrid = pl.cdiv(n, cblk)
    f32 = jnp.float32
    bf16 = jnp.bfloat16

    h, qa, qbt, ve1 = pl.pallas_call(
        _gc1_kernel,
        grid=(ggrid,),
        in_specs=[pl.BlockSpec((blk, n), lambda i: (i, 0)),
                  pl.BlockSpec((n, nfeat), lambda i: (0, 0)),
                  pl.BlockSpec((nfeat, nhid), lambda i: (0, 0)),
                  pl.BlockSpec((1, nhid), lambda i: (0, 0)),
                  pl.BlockSpec((nhid, nhid), lambda i: (0, 0)),
                  pl.BlockSpec((nhid, nhid), lambda i: (0, 0))],
        out_specs=[pl.BlockSpec((blk, nhid), lambda i: (i, 0)),
                   pl.BlockSpec((blk, nhid), lambda i: (i, 0)),
                   pl.BlockSpec((nhid, blk), lambda i: (0, i)),
                   pl.BlockSpec((blk, nhid + 1), lambda i: (i, 0))],
        out_shape=[jax.ShapeDtypeStruct((n, nhid), f32),
                   jax.ShapeDtypeStruct((n, nhid), bf16),
                   jax.ShapeDtypeStruct((nhid, n), bf16),
                   jax.ShapeDtypeStruct((n, nhid + 1), jnp.float8_e4m3fn)],
        scratch_shapes=[pltpu.VMEM((n, nhid), f32)],
        compiler_params=_ARBITRARY,
    )(adj, x, W1, b1.reshape(1, nhid), Wa, Wb)

    ab = (jnp.stack([alpha, beta]) / (alpha + beta)).reshape(1, 2)
    f8 = jnp.float8_e4m3fn

    ve2, p = pl.pallas_call(
        _crf1_kernel,
        grid=(cgrid,),
        in_specs=[pl.BlockSpec((cblk, nhid), lambda i: (i, 0)),
                  pl.BlockSpec((nhid, n), lambda i: (0, 0)),
                  pl.BlockSpec((n, nhid + 1), lambda i: (0, 0)),
                  pl.BlockSpec((cblk, nhid), lambda i: (i, 0)),
                  pl.BlockSpec((1, 2), lambda i: (0, 0))],
        out_specs=[pl.BlockSpec((cblk, nhid + 1), lambda i: (i, 0)),
                   pl.BlockSpec((cblk, n), lambda i: (i, 0))],
        out_shape=[jax.ShapeDtypeStruct((n, nhid + 1), f8),
                   jax.ShapeDtypeStruct((n, n), f8)],
        compiler_params=_PARALLEL,
    )(qa, qbt, ve1, h, ab)

    t2 = pl.pallas_call(
        _crf2_kernel,
        grid=(cgrid,),
        in_specs=[pl.BlockSpec((cblk, n), lambda i: (i, 0)),
                  pl.BlockSpec((n, nhid + 1), lambda i: (0, 0)),
                  pl.BlockSpec((cblk, nhid), lambda i: (i, 0)),
                  pl.BlockSpec((1, 2), lambda i: (0, 0)),
                  pl.BlockSpec((nhid, nclass), lambda i: (0, 0))],
        out_specs=pl.BlockSpec((cblk, nclass), lambda i: (i, 0)),
        out_shape=jax.ShapeDtypeStruct((n, nclass), f32),
        compiler_params=_PARALLEL,
    )(p, ve2, h, ab, W2)

    blk2 = 512 if n >= 512 else blk
    out = pl.pallas_call(
        _gc2_kernel,
        grid=(pl.cdiv(n, blk2),),
        in_specs=[pl.BlockSpec((blk2, n), lambda i: (i, 0)),
                  pl.BlockSpec((n, nclass), lambda i: (0, 0)),
                  pl.BlockSpec((1, nclass), lambda i: (0, 0))],
        out_specs=pl.BlockSpec((blk2, nclass), lambda i: (i, 0)),
        out_shape=jax.ShapeDtypeStruct((n, nclass), f32),
        compiler_params=_PARALLEL,
    )(adj, t2, b2.reshape(1, nclass))
    return out


# speedup vs baseline: 1.0181x; 1.0181x over previous
import jax
import jax.numpy as jnp
from jax.experimental import pallas as pl
from jax.experimental.pallas import tpu as pltpu

_PARALLEL = pltpu.CompilerParams(dimension_semantics=("parallel",))
_ARBITRARY = pltpu.CompilerParams(dimension_semantics=("arbitrary",))
_HALF = 256


def _pick_blk(m, pref):
    for b in (pref, 200, 100, 50, 25, 10, 8, 5, 4, 2, 1):
        if m % b == 0:
            return b
    return m


def _gc1_kernel(adj_ref, x_ref, w1_ref, b_ref, wa_ref, wb_ref,
                h_ref, qa_ref, qbt_ref, ve_ref, t_ref):
    @pl.when(pl.program_id(0) == 0)
    def _():
        t_ref[...] = jnp.dot(x_ref[...], w1_ref[...],
                             preferred_element_type=jnp.float32)

    acc = jnp.dot(adj_ref[...], t_ref[...],
                  preferred_element_type=jnp.float32)
    h = jnp.maximum(acc + b_ref[...], 0.0)
    h_ref[...] = h
    qa_ref[...] = jnp.dot(h, wa_ref[...],
                          preferred_element_type=jnp.float32
                          ).astype(jnp.bfloat16)
    qb = jnp.dot(h, wb_ref[...], preferred_element_type=jnp.float32)
    qbt_ref[...] = qb.T.astype(jnp.bfloat16)
    nhid = h_ref.shape[1]
    ve_ref[:, :nhid] = h.astype(ve_ref.dtype)
    ve_ref[:, nhid:] = jnp.ones_like(ve_ref[:, nhid:])


def _crf1_kernel(qa_ref, qbt_ref, ve_ref, h_ref, ab_ref, ve2_ref, p_ref):
    qbt = qbt_ref[...]
    ve = ve_ref[...]
    nhid = h_ref.shape[1]
    a = ab_ref[0, 0]
    b = ab_ref[0, 1]
    chain = _HALF if qa_ref.shape[0] % _HALF == 0 else qa_ref.shape[0]
    for half in range(qa_ref.shape[0] // chain):
        sl = pl.ds(half * chain, chain)
        logits = jnp.dot(qa_ref[sl, :], qbt,
                         preferred_element_type=jnp.float32
                         ).astype(jnp.bfloat16)
        m = jnp.max(logits, axis=1, keepdims=True)
        p = jnp.exp(logits - m)
        p_ref[sl, :] = p.astype(p_ref.dtype)
        num = jnp.dot(p, ve, preferred_element_type=jnp.float32)
        s = num[:, nhid:nhid + 1]
        ve2_ref[sl, :nhid] = ((a * h_ref[sl, :] + b * (num[:, :nhid] / s))
                              * 0.0625).astype(ve2_ref.dtype)
    ve2_ref[:, nhid:] = jnp.full_like(ve2_ref[:, nhid:], 0.0625)


def _crf2_kernel(p_ref, ve_ref, h_ref, ab_ref, w2_ref, t2_ref):
    num = jnp.dot(p_ref[...], ve_ref[...],
                  preferred_element_type=jnp.float32)
    nhid = h_ref.shape[1]
    s = num[:, nhid:nhid + 1]
    a = ab_ref[0, 0]
    b = ab_ref[0, 1]
    h2 = a * h_ref[...] + b * (num[:, :nhid] / s)
    t2_ref[...] = jnp.dot(h2, w2_ref[...],
                          preferred_element_type=jnp.float32)


def _gc2_kernel(adj_ref, t2_ref, b_ref, out_ref):
    logits = jnp.dot(adj_ref[...], t2_ref[...],
                     preferred_element_type=jnp.float32) + b_ref[...]
    m = jnp.max(logits, axis=1, keepdims=True)
    ls = logits - m
    lse = jnp.log(jnp.sum(jnp.exp(ls), axis=1, keepdims=True))
    out_ref[...] = ls - lse


def kernel(x, adj, W1, b1, W2, b2, Wa, Wb, alpha, beta):
    n, nfeat = x.shape
    nhid = W1.shape[1]
    nclass = W2.shape[1]
    blk = 256 if n >= 256 else _pick_blk(n, 8)
    ggrid = pl.cdiv(n, blk)
    cblk = 2 * _HALF if n >= 2 * _HALF else _pick_blk(n, 8)
    cgrid = pl.cdiv(n, cblk)
    f32 = jnp.float32
    bf16 = jnp.bfloat16

    h, qa, qbt, ve1 = pl.pallas_call(
        _gc1_kernel,
        grid=(ggrid,),
        in_specs=[pl.BlockSpec((blk, n), lambda i: (i, 0)),
                  pl.BlockSpec((n, nfeat), lambda i: (0, 0)),
                  pl.BlockSpec((nfeat, nhid), lambda i: (0, 0)),
                  pl.BlockSpec((1, nhid), lambda i: (0, 0)),
                  pl.BlockSpec((nhid, nhid), lambda i: (0, 0)),
                  pl.BlockSpec((nhid, nhid), lambda i: (0, 0))],
        out_specs=[pl.BlockSpec((blk, nhid), lambda i: (i, 0)),
                   pl.BlockSpec((blk, nhid), lambda i: (i, 0)),
                   pl.BlockSpec((nhid, blk), lambda i: (0, i)),
                   pl.BlockSpec((blk, nhid + 1), lambda i: (i, 0))],
        out_shape=[jax.ShapeDtypeStruct((n, nhid), f32),
                   jax.ShapeDtypeStruct((n, nhid), bf16),
                   jax.ShapeDtypeStruct((nhid, n), bf16),
                   jax.ShapeDtypeStruct((n, nhid + 1), bf16)],
        scratch_shapes=[pltpu.VMEM((n, nhid), f32)],
        compiler_params=_ARBITRARY,
    )(adj, x, W1, b1.reshape(1, nhid), Wa, Wb)

    ab = (jnp.stack([alpha, beta]) / (alpha + beta)).reshape(1, 2)
    f8 = jnp.float8_e4m3fn

    ve2, p = pl.pallas_call(
        _crf1_kernel,
        grid=(cgrid,),
        in_specs=[pl.BlockSpec((cblk, nhid), lambda i: (i, 0)),
                  pl.BlockSpec((nhid, n), lambda i: (0, 0)),
                  pl.BlockSpec((n, nhid + 1), lambda i: (0, 0)),
                  pl.BlockSpec((cblk, nhid), lambda i: (i, 0)),
                  pl.BlockSpec((1, 2), lambda i: (0, 0))],
        out_specs=[pl.BlockSpec((cblk, nhid + 1), lambda i: (i, 0)),
                   pl.BlockSpec((cblk, n), lambda i: (i, 0))],
        out_shape=[jax.ShapeDtypeStruct((n, nhid + 1), f8),
                   jax.ShapeDtypeStruct((n, n), f8)],
        compiler_params=_PARALLEL,
    )(qa, qbt, ve1, h, ab)

    t2 = pl.pallas_call(
        _crf2_kernel,
        grid=(cgrid,),
        in_specs=[pl.BlockSpec((cblk, n), lambda i: (i, 0)),
                  pl.BlockSpec((n, nhid + 1), lambda i: (0, 0)),
                  pl.BlockSpec((cblk, nhid), lambda i: (i, 0)),
                  pl.BlockSpec((1, 2), lambda i: (0, 0)),
                  pl.BlockSpec((nhid, nclass), lambda i: (0, 0))],
        out_specs=pl.BlockSpec((cblk, nclass), lambda i: (i, 0)),
        out_shape=jax.ShapeDtypeStruct((n, nclass), f32),
        compiler_params=_PARALLEL,
    )(p, ve2, h, ab, W2)

    blk2 = 512 if n >= 512 else blk
    out = pl.pallas_call(
        _gc2_kernel,
        grid=(pl.cdiv(n, blk2),),
        in_specs=[pl.BlockSpec((blk2, n), lambda i: (i, 0)),
                  pl.BlockSpec((n, nclass), lambda i: (0, 0)),
                  pl.BlockSpec((1, nclass), lambda i: (0, 0))],
        out_specs=pl.BlockSpec((blk2, nclass), lambda i: (i, 0)),
        out_shape=jax.ShapeDtypeStruct((n, nclass), f32),
        compiler_params=_PARALLEL,
    )(adj, t2, b2.reshape(1, nclass))
    return out


# speedup vs baseline: 1.0923x; 1.0729x over previous
import jax
import jax.numpy as jnp
from jax.experimental import pallas as pl
from jax.experimental.pallas import tpu as pltpu

_PARALLEL = pltpu.CompilerParams(dimension_semantics=("parallel",))
_ARBITRARY = pltpu.CompilerParams(dimension_semantics=("arbitrary",))
_HALF = 256


def _pick_blk(m, pref):
    for b in (pref, 200, 100, 50, 25, 10, 8, 5, 4, 2, 1):
        if m % b == 0:
            return b
    return m


def _gc1_kernel(adj_ref, x_ref, w1_ref, b_ref, wa_ref, wb_ref,
                h_ref, qa_ref, qbt_ref, ve_ref, t_ref):
    @pl.when(pl.program_id(0) == 0)
    def _():
        t_ref[...] = jnp.dot(x_ref[...], w1_ref[...],
                             preferred_element_type=jnp.float32)

    acc = jnp.dot(adj_ref[...], t_ref[...],
                  preferred_element_type=jnp.float32)
    h = jnp.maximum(acc + b_ref[...], 0.0)
    h_ref[...] = h
    qa_ref[...] = jnp.dot(h, wa_ref[...],
                          preferred_element_type=jnp.float32
                          ).astype(qa_ref.dtype)
    qb = jnp.dot(h, wb_ref[...], preferred_element_type=jnp.float32)
    qbt_ref[...] = qb.T.astype(qbt_ref.dtype)
    nhid = h_ref.shape[1]
    ve_ref[:, :nhid] = h.astype(ve_ref.dtype)
    ve_ref[:, nhid:] = jnp.ones_like(ve_ref[:, nhid:])


def _crf1_kernel(qa_ref, qbt_ref, ve_ref, h_ref, ab_ref, ve2_ref, p_ref):
    qbt = qbt_ref[...]
    ve = ve_ref[...]
    nhid = h_ref.shape[1]
    a = ab_ref[0, 0]
    b = ab_ref[0, 1]
    chain = _HALF if qa_ref.shape[0] % _HALF == 0 else qa_ref.shape[0]
    for half in range(qa_ref.shape[0] // chain):
        sl = pl.ds(half * chain, chain)
        logits = jnp.dot(qa_ref[sl, :], qbt,
                         preferred_element_type=jnp.float32
                         ).astype(jnp.bfloat16)
        m = jnp.max(logits, axis=1, keepdims=True)
        p = jnp.exp(logits - m)
        p_ref[sl, :] = p.astype(p_ref.dtype)
        num = jnp.dot(p, ve, preferred_element_type=jnp.float32)
        s = num[:, nhid:nhid + 1]
        ve2_ref[sl, :nhid] = ((a * h_ref[sl, :] + b * (num[:, :nhid] / s))
                              * 0.0625).astype(ve2_ref.dtype)
    ve2_ref[:, nhid:] = jnp.full_like(ve2_ref[:, nhid:], 0.0625)


def _crf2_kernel(p_ref, ve_ref, h_ref, ab_ref, w2_ref, t2_ref):
    num = jnp.dot(p_ref[...], ve_ref[...],
                  preferred_element_type=jnp.float32)
    nhid = h_ref.shape[1]
    s = num[:, nhid:nhid + 1]
    a = ab_ref[0, 0]
    b = ab_ref[0, 1]
    h2 = a * h_ref[...] + b * (num[:, :nhid] / s)
    t2_ref[...] = jnp.dot(h2, w2_ref[...],
                          preferred_element_type=jnp.float32)


def _gc2_kernel(adj_ref, t2_ref, b_ref, out_ref):
    logits = jnp.dot(adj_ref[...], t2_ref[...],
                     preferred_element_type=jnp.float32) + b_ref[...]
    m = jnp.max(logits, axis=1, keepdims=True)
    ls = logits - m
    lse = jnp.log(jnp.sum(jnp.exp(ls), axis=1, keepdims=True))
    out_ref[...] = ls - lse


def kernel(x, adj, W1, b1, W2, b2, Wa, Wb, alpha, beta):
    n, nfeat = x.shape
    nhid = W1.shape[1]
    nclass = W2.shape[1]
    blk = 256 if n >= 256 else _pick_blk(n, 8)
    ggrid = pl.cdiv(n, blk)
    cblk = 2 * _HALF if n >= 2 * _HALF else _pick_blk(n, 8)
    cgrid = pl.cdiv(n, cblk)
    f32 = jnp.float32
    bf16 = jnp.bfloat16

    h, qa, qbt, ve1 = pl.pallas_call(
        _gc1_kernel,
        grid=(ggrid,),
        in_specs=[pl.BlockSpec((blk, n), lambda i: (i, 0)),
                  pl.BlockSpec((n, nfeat), lambda i: (0, 0)),
                  pl.BlockSpec((nfeat, nhid), lambda i: (0, 0)),
                  pl.BlockSpec((1, nhid), lambda i: (0, 0)),
                  pl.BlockSpec((nhid, nhid), lambda i: (0, 0)),
                  pl.BlockSpec((nhid, nhid), lambda i: (0, 0))],
        out_specs=[pl.BlockSpec((blk, nhid), lambda i: (i, 0)),
                   pl.BlockSpec((blk, nhid), lambda i: (i, 0)),
                   pl.BlockSpec((nhid, blk), lambda i: (0, i)),
                   pl.BlockSpec((blk, nhid + 1), lambda i: (i, 0))],
        out_shape=[jax.ShapeDtypeStruct((n, nhid), f32),
                   jax.ShapeDtypeStruct((n, nhid), jnp.float8_e4m3fn),
                   jax.ShapeDtypeStruct((nhid, n), jnp.float8_e4m3fn),
                   jax.ShapeDtypeStruct((n, nhid + 1), bf16)],
        scratch_shapes=[pltpu.VMEM((n, nhid), f32)],
        compiler_params=_ARBITRARY,
    )(adj, x, W1, b1.reshape(1, nhid), Wa, Wb)

    ab = (jnp.stack([alpha, beta]) / (alpha + beta)).reshape(1, 2)
    f8 = jnp.float8_e4m3fn

    ve2, p = pl.pallas_call(
        _crf1_kernel,
        grid=(cgrid,),
        in_specs=[pl.BlockSpec((cblk, nhid), lambda i: (i, 0)),
                  pl.BlockSpec((nhid, n), lambda i: (0, 0)),
                  pl.BlockSpec((n, nhid + 1), lambda i: (0, 0)),
                  pl.BlockSpec((cblk, nhid), lambda i: (i, 0)),
                  pl.BlockSpec((1, 2), lambda i: (0, 0))],
        out_specs=[pl.BlockSpec((cblk, nhid + 1), lambda i: (i, 0)),
                   pl.BlockSpec((cblk, n), lambda i: (i, 0))],
        out_shape=[jax.ShapeDtypeStruct((n, nhid + 1), f8),
                   jax.ShapeDtypeStruct((n, n), f8)],
        compiler_params=_PARALLEL,
    )(qa, qbt, ve1, h, ab)

    t2 = pl.pallas_call(
        _crf2_kernel,
        grid=(cgrid,),
        in_specs=[pl.BlockSpec((cblk, n), lambda i: (i, 0)),
                  pl.BlockSpec((n, nhid + 1), lambda i: (0, 0)),
                  pl.BlockSpec((cblk, nhid), lambda i: (i, 0)),
                  pl.BlockSpec((1, 2), lambda i: (0, 0)),
                  pl.BlockSpec((nhid, nclass), lambda i: (0, 0))],
        out_specs=pl.BlockSpec((cblk, nclass), lambda i: (i, 0)),
        out_shape=jax.ShapeDtypeStruct((n, nclass), f32),
        compiler_params=_PARALLEL,
    )(p, ve2, h, ab, W2)

    blk2 = 512 if n >= 512 else blk
    out = pl.pallas_call(
        _gc2_kernel,
        grid=(pl.cdiv(n, blk2),),
        in_specs=[pl.BlockSpec((blk2, n), lambda i: (i, 0)),
                  pl.BlockSpec((n, nclass), lambda i: (0, 0)),
                  pl.BlockSpec((1, nclass), lambda i: (0, 0))],
        out_specs=pl.BlockSpec((blk2, nclass), lambda i: (i, 0)),
        out_shape=jax.ShapeDtypeStruct((n, nclass), f32),
        compiler_params=_PARALLEL,
    )(adj, t2, b2.reshape(1, nclass))
    return out
